# baseline (speedup 1.0000x reference)
import functools

import jax
import jax.numpy as jnp
from jax import lax
from jax.experimental import pallas as pl
from jax.experimental.pallas import tpu as pltpu
from jax.experimental.pallas import tpu_sc as plsc

NC = 2
NS = 16
NW = NC * NS
CHUNK = 128


def _mm_bias_block(x_ref, w_ref, b_ref, o_ref):
    o_ref[...] = (
        jnp.dot(x_ref[...], w_ref[...], preferred_element_type=jnp.float32)
        + b_ref[...]
    )


@functools.lru_cache(maxsize=None)
def _make_mm_bias(n, k, h, bm):
    return pl.pallas_call(
        _mm_bias_block,
        grid=(n // bm,),
        in_specs=[
            pl.BlockSpec((bm, k), lambda i: (i, 0)),
            pl.BlockSpec((k, h), lambda i: (0, 0)),
            pl.BlockSpec((1, h), lambda i: (0, 0)),
        ],
        out_specs=pl.BlockSpec((bm, h), lambda i: (i, 0)),
        out_shape=jax.ShapeDtypeStruct((n, h), jnp.float32),
    )


def _mm_split_block(relu_in, hh, x_ref, w_ref, b_ref, o_ref):
    x = x_ref[...]
    if relu_in:
        x = jnp.maximum(x, 0.0)
    r = jnp.dot(x, w_ref[...], preferred_element_type=jnp.float32) + b_ref[...]
    o_ref[0] = r[:, :hh]
    o_ref[1] = r[:, hh:]


@functools.lru_cache(maxsize=None)
def _make_mm_split(n, k, h, bm, relu_in):
    hh = h // 2
    return pl.pallas_call(
        functools.partial(_mm_split_block, relu_in, hh),
        grid=(n // bm,),
        in_specs=[
            pl.BlockSpec((bm, k), lambda i: (i, 0)),
            pl.BlockSpec((k, h), lambda i: (0, 0)),
            pl.BlockSpec((1, h), lambda i: (0, 0)),
        ],
        out_specs=pl.BlockSpec((2, bm, hh), lambda i: (0, i, 0)),
        out_shape=jax.ShapeDtypeStruct((2, n, hh), jnp.float32),
    )


def _fused_mm_split_block(hh, p_ref, q_ref, w_ref, b_ref, o_ref):
    a = jnp.maximum(p_ref[...] + q_ref[...], 0.0)
    r = jnp.dot(a, w_ref[...], preferred_element_type=jnp.float32) + b_ref[...]
    o_ref[0] = r[:, :hh]
    o_ref[1] = r[:, hh:]


@functools.lru_cache(maxsize=None)
def _make_fused_mm_split(n, k, h, bm):
    hh = h // 2
    nb = n // bm
    return pl.pallas_call(
        functools.partial(_fused_mm_split_block, hh),
        grid=(nb,),
        in_specs=[
            pl.BlockSpec((bm, k), lambda i: (i, 0)),
            pl.BlockSpec((bm, k), lambda i, _nb=nb: (i + _nb, 0)),
            pl.BlockSpec((k, h), lambda i: (0, 0)),
            pl.BlockSpec((1, h), lambda i: (0, 0)),
        ],
        out_specs=pl.BlockSpec((2, bm, hh), lambda i: (0, i, 0)),
        out_shape=jax.ShapeDtypeStruct((2, n, hh), jnp.float32),
    )


@functools.lru_cache(maxsize=None)
def _make_sc_agg_edge(n_nodes, n_chunks, d, nbuf):
    cpw = -(-n_chunks // NW)
    slab = (n_nodes // NS) // 8 * 8
    rem = n_nodes - NS * slab
    assert rem % 8 == 0

    mesh = plsc.VectorSubcoreMesh(core_axis_name="c", subcore_axis_name="s")
    scratch = [
        pltpu.VMEM((cpw, CHUNK), jnp.int32),
        pltpu.VMEM((cpw, CHUNK), jnp.int32),
        pltpu.VMEM((nbuf, CHUNK, d), jnp.float32),
        pltpu.VMEM_SHARED((n_nodes, d), jnp.float32),
    ] + [pltpu.SemaphoreType.DMA] * nbuf

    @functools.partial(
        pl.kernel,
        out_type=jax.ShapeDtypeStruct((NC * n_nodes, d), jnp.float32),
        mesh=mesh,
        scratch_types=scratch,
    )
    def agg(h_hbm, src_hbm, dst_hbm, zeros_hbm, out_hbm,
            src_v, dst_v, rows_v, acc_sh, *sems):
        c = lax.axis_index("c")
        s = lax.axis_index("s")
        wid = s * NC + c
        row0 = s * slab
        c0 = wid * cpw
        cpw_eff = jnp.clip(n_chunks - c0, 0, cpw)

        pltpu.sync_copy(src_hbm.at[pl.ds(c0, cpw)], src_v)
        pltpu.sync_copy(dst_hbm.at[pl.ds(c0, cpw)], dst_v)

        for b in range(nbuf):
            @pl.when(b < cpw_eff)
            def _():
                pltpu.async_copy(h_hbm.at[src_v.at[b]], rows_v.at[b], sems[b])

        pltpu.sync_copy(zeros_hbm.at[pl.ds(row0, slab)],
                        acc_sh.at[pl.ds(row0, slab)])
        if rem:
            @pl.when(s == NS - 1)
            def _():
                pltpu.sync_copy(zeros_hbm.at[pl.ds(NS * slab, rem)],
                                acc_sh.at[pl.ds(NS * slab, rem)])
        plsc.subcore_barrier()

        def _wait(b):
            pltpu.make_async_copy(
                h_hbm.at[pl.ds(0, CHUNK)], rows_v.at[b], sems[b]).wait()

        n_groups = -(-cpw // nbuf)

        def step(g, carry):
            for b in range(nbuf):
                chunk = g * nbuf + b
                nxt = (g + 1) * nbuf + b

                @pl.when(chunk < cpw_eff)
                def _():
                    _wait(b)
                    pltpu.sync_copy(rows_v.at[b], acc_sh.at[dst_v.at[chunk]],
                                    add=True)

                @pl.when(nxt < cpw_eff)
                def _():
                    pltpu.async_copy(h_hbm.at[src_v.at[nxt]],
                                     rows_v.at[b], sems[b])
            return carry

        lax.fori_loop(0, n_groups, step, 0)

        plsc.subcore_barrier()
        pltpu.sync_copy(acc_sh.at[pl.ds(row0, slab)],
                        out_hbm.at[pl.ds(c * n_nodes + row0, slab)])
        if rem:
            @pl.when(s == NS - 1)
            def _():
                pltpu.sync_copy(
                    acc_sh.at[pl.ds(NS * slab, rem)],
                    out_hbm.at[pl.ds(c * n_nodes + NS * slab, rem)])

    return agg


@functools.lru_cache(maxsize=None)
def _make_sc_agg_feat(n_nodes, n_chunks, dh, nbuf, stage_table=False):
    cpw = -(-n_chunks // NS)
    slab = (n_nodes // NS) // 8 * 8
    rem = n_nodes - NS * slab
    assert rem % 8 == 0

    mesh = plsc.VectorSubcoreMesh(core_axis_name="c", subcore_axis_name="s")
    scratch = [
        pltpu.VMEM((cpw, CHUNK), jnp.int32),
        pltpu.VMEM((cpw, CHUNK), jnp.int32),
        pltpu.VMEM((nbuf, CHUNK, dh), jnp.float32),
        pltpu.VMEM_SHARED((n_nodes, dh), jnp.float32),
    ] + ([pltpu.VMEM_SHARED((n_nodes, dh), jnp.float32)] if stage_table else []
         ) + [pltpu.SemaphoreType.DMA] * nbuf

    @functools.partial(
        pl.kernel,
        out_type=jax.ShapeDtypeStruct((n_nodes, 2 * dh), jnp.float32),
        mesh=mesh,
        scratch_types=scratch,
        compiler_params=pltpu.CompilerParams(use_tc_tiling_on_sc=False),
    )
    def agg(h_hbm, src_hbm, dst_hbm, zeros_hbm, out_hbm,
            src_v, dst_v, rows_v, acc_sh, *rest):
        if stage_table:
            table_sh, *sems = rest
        else:
            sems = rest
        c = lax.axis_index("c")
        s = lax.axis_index("s")
        row0 = s * slab
        c0 = s * cpw
        hc = h_hbm.at[c]
        cpw_eff = jnp.clip(n_chunks - c0, 0, cpw)

        pltpu.sync_copy(src_hbm.at[pl.ds(c0, cpw)], src_v)
        pltpu.sync_copy(dst_hbm.at[pl.ds(c0, cpw)], dst_v)

        if stage_table:
            pltpu.sync_copy(hc.at[pl.ds(row0, slab)],
                            table_sh.at[pl.ds(row0, slab)])
            if rem:
                @pl.when(s == NS - 1)
                def _():
                    pltpu.sync_copy(hc.at[pl.ds(NS * slab, rem)],
                                    table_sh.at[pl.ds(NS * slab, rem)])
            src_tab = table_sh
        else:
            src_tab = hc

        pltpu.sync_copy(zeros_hbm.at[pl.ds(row0, slab)],
                        acc_sh.at[pl.ds(row0, slab)])
        if rem:
            @pl.when(s == NS - 1)
            def _():
                pltpu.sync_copy(zeros_hbm.at[pl.ds(NS * slab, rem)],
                                acc_sh.at[pl.ds(NS * slab, rem)])
        plsc.subcore_barrier()

        for b in range(nbuf):
            @pl.when(b < cpw_eff)
            def _():
                pltpu.async_copy(src_tab.at[src_v.at[b]], rows_v.at[b],
                                 sems[b])

        def _wait(b):
            pltpu.make_async_copy(
                hc.at[pl.ds(0, CHUNK)], rows_v.at[b], sems[b]).wait()

        n_groups = -(-cpw // nbuf)

        def step(g, carry):
            for b in range(nbuf):
                chunk = g * nbuf + b
                nxt = (g + 1) * nbuf + b

                @pl.when(chunk < cpw_eff)
                def _():
                    _wait(b)
                    pltpu.sync_copy(rows_v.at[b], acc_sh.at[dst_v.at[chunk]],
                                    add=True)

                @pl.when(nxt < cpw_eff)
                def _():
                    pltpu.async_copy(src_tab.at[src_v.at[nxt]],
                                     rows_v.at[b], sems[b])
            return carry

        lax.fori_loop(0, n_groups, step, 0)

        plsc.subcore_barrier()
        pltpu.sync_copy(acc_sh.at[pl.ds(row0, slab)],
                        out_hbm.at[pl.ds(row0, slab), pl.ds(c * dh, dh)])
        if rem:
            @pl.when(s == NS - 1)
            def _():
                pltpu.sync_copy(
                    acc_sh.at[pl.ds(NS * slab, rem)],
                    out_hbm.at[pl.ds(NS * slab, rem), pl.ds(c * dh, dh)])

    return agg


def kernel(x, adj, W1, b1, W2, b2):
    n, f = x.shape
    h1w = W1.shape[1]
    h2w = W2.shape[1]
    e = adj.shape[1]
    n_chunks = e // CHUNK
    pad_chunks = -(-n_chunks // NW) * NW
    idx = adj.astype(jnp.int32).reshape(2, n_chunks, CHUNK)
    idx = jnp.pad(idx, ((0, 0), (0, pad_chunks - n_chunks), (0, 0)))
    src, dst = idx[0], idx[1]

    bm = 2000
    h1 = _make_mm_split(n, f, h1w, bm, False)(x, W1, b1.reshape(1, h1w))
    z1 = jnp.zeros((n, h1w // 2), jnp.float32)
    g1 = _make_sc_agg_feat(n, n_chunks, h1w // 2, 6)(h1, src, dst, z1)
    z2 = _make_mm_split(n, h1w, h2w, bm, True)(g1, W2, b2.reshape(1, h2w))
    zz = jnp.zeros((n, h2w // 2), jnp.float32)
    return _make_sc_agg_feat(n, n_chunks, h2w // 2, 6, True)(z2, src, dst, zz)


# speedup vs baseline: 1.1004x; 1.1004x over previous
import functools

import jax
import jax.numpy as jnp
from jax import lax
from jax.experimental import pallas as pl
from jax.experimental.pallas import tpu as pltpu
from jax.experimental.pallas import tpu_sc as plsc

NC = 2
NS = 16
NW = NC * NS
CHUNK = 128


def _mm_split_block(relu_in, hh, x_ref, w_ref, b_ref, o_ref):
    x = x_ref[...]
    if relu_in:
        x = jnp.maximum(x, 0.0)
    r = jnp.dot(x, w_ref[...], preferred_element_type=jnp.float32) + b_ref[...]
    o_ref[0] = r[:, :hh]
    o_ref[1] = r[:, hh:]


@functools.lru_cache(maxsize=None)
def _make_mm_split(n, k, h, bm, relu_in):
    hh = h // 2
    return pl.pallas_call(
        functools.partial(_mm_split_block, relu_in, hh),
        grid=(n // bm,),
        in_specs=[
            pl.BlockSpec((bm, k), lambda i: (i, 0)),
            pl.BlockSpec((k, h), lambda i: (0, 0)),
            pl.BlockSpec((1, h), lambda i: (0, 0)),
        ],
        out_specs=pl.BlockSpec((2, bm, hh), lambda i: (0, i, 0)),
        out_shape=jax.ShapeDtypeStruct((2, n, hh), jnp.float32),
    )


@functools.lru_cache(maxsize=None)
def _make_sc_agg_feat(n_nodes, n_chunks, dh, nbuf):
    cpw = -(-n_chunks // NS)
    slab = (n_nodes // NS) // 8 * 8
    rem = n_nodes - NS * slab
    assert rem % 8 == 0

    mesh = plsc.VectorSubcoreMesh(core_axis_name="c", subcore_axis_name="s")
    scratch = [
        pltpu.VMEM((cpw, CHUNK), jnp.int32),
        pltpu.VMEM((cpw, CHUNK), jnp.int32),
        pltpu.VMEM((nbuf, CHUNK, dh), jnp.float32),
        pltpu.VMEM_SHARED((n_nodes, dh), jnp.float32),
    ] + [pltpu.SemaphoreType.DMA] * nbuf

    @functools.partial(
        pl.kernel,
        out_type=jax.ShapeDtypeStruct((n_nodes, 2 * dh), jnp.float32),
        mesh=mesh,
        scratch_types=scratch,
        compiler_params=pltpu.CompilerParams(use_tc_tiling_on_sc=False),
    )
    def agg(h_hbm, src_hbm, dst_hbm, zeros_hbm, out_hbm,
            src_v, dst_v, rows_v, acc_sh, *sems):
        c = lax.axis_index("c")
        s = lax.axis_index("s")
        row0 = s * slab
        c0 = s * cpw
        hc = h_hbm.at[c]
        cpw_eff = jnp.clip(n_chunks - c0, 0, cpw)

        pltpu.sync_copy(src_hbm.at[pl.ds(c0, cpw)], src_v)
        pltpu.sync_copy(dst_hbm.at[pl.ds(c0, cpw)], dst_v)

        for b in range(nbuf):
            @pl.when(b < cpw_eff)
            def _():
                pltpu.async_copy(hc.at[src_v.at[b]], rows_v.at[b], sems[b])

        pltpu.sync_copy(zeros_hbm.at[pl.ds(row0, slab)],
                        acc_sh.at[pl.ds(row0, slab)])
        if rem:
            @pl.when(s == NS - 1)
            def _():
                pltpu.sync_copy(zeros_hbm.at[pl.ds(NS * slab, rem)],
                                acc_sh.at[pl.ds(NS * slab, rem)])
        plsc.subcore_barrier()

        def _wait(b):
            pltpu.make_async_copy(
                hc.at[pl.ds(0, CHUNK)], rows_v.at[b], sems[b]).wait()

        n_groups = -(-cpw // nbuf)

        def step(g, carry):
            for b in range(nbuf):
                chunk = g * nbuf + b
                nxt = (g + 1) * nbuf + b

                @pl.when(chunk < cpw_eff)
                def _():
                    _wait(b)
                    pltpu.sync_copy(rows_v.at[b], acc_sh.at[dst_v.at[chunk]],
                                    add=True)

                @pl.when(nxt < cpw_eff)
                def _():
                    pltpu.async_copy(hc.at[src_v.at[nxt]],
                                     rows_v.at[b], sems[b])
            return carry

        lax.fori_loop(0, n_groups, step, 0)

        plsc.subcore_barrier()
        pltpu.sync_copy(acc_sh.at[pl.ds(row0, slab)],
                        out_hbm.at[pl.ds(row0, slab), pl.ds(c * dh, dh)])
        if rem:
            @pl.when(s == NS - 1)
            def _():
                pltpu.sync_copy(
                    acc_sh.at[pl.ds(NS * slab, rem)],
                    out_hbm.at[pl.ds(NS * slab, rem), pl.ds(c * dh, dh)])

    return agg


def kernel(x, adj, W1, b1, W2, b2):
    n, f = x.shape
    h1w = W1.shape[1]
    h2w = W2.shape[1]
    e = adj.shape[1]
    n_chunks = e // CHUNK
    pad_chunks = -(-n_chunks // NW) * NW
    idx = adj.astype(jnp.int32).reshape(2, n_chunks, CHUNK)
    idx = jnp.pad(idx, ((0, 0), (0, pad_chunks - n_chunks), (0, 0)))
    src, dst = idx[0], idx[1]

    bm = 2000
    h1 = _make_mm_split(n, f, h1w, bm, False)(x, W1, b1.reshape(1, h1w))
    z1 = jnp.zeros((n, h1w // 2), jnp.float32)
    g1 = _make_sc_agg_feat(n, n_chunks, h1w // 2, 6)(h1, src, dst, z1)
    z2 = _make_mm_split(n, h1w, h2w, bm, True)(g1, W2, b2.reshape(1, h2w))
    zz = jnp.zeros((n, h2w // 2), jnp.float32)
    return _make_sc_agg_feat(n, n_chunks, h2w // 2, 6)(z2, src, dst, zz)
